# initial kernel scaffold (unmeasured)
import jax
import jax.numpy as jnp
from jax import lax
from jax.experimental import pallas as pl
from jax.experimental.pallas import tpu as pltpu


def kernel(x, pi):
    shard_shape = x.shape

    def body(pi_ref, x_ref, out_ref, send_sem, recv_sem, local_sem):
        my_x = lax.axis_index("x")
        my_y = lax.axis_index("y")
        dest = pi_ref[my_x]

        @pl.when(dest != my_x)
        def _swap():
            rdma = pltpu.make_async_remote_copy(
                src_ref=x_ref,
                dst_ref=out_ref,
                send_sem=send_sem,
                recv_sem=recv_sem,
                device_id=(dest, my_y),
                device_id_type=pl.DeviceIdType.MESH,
            )
            rdma.start()
            rdma.wait()

        @pl.when(dest == my_x)
        def _local():
            copy = pltpu.make_async_copy(x_ref, out_ref, local_sem)
            copy.start()
            copy.wait()

    return pl.pallas_call(
        body,
        out_shape=jax.ShapeDtypeStruct(shard_shape, jnp.float32),
        in_specs=[
            pl.BlockSpec(memory_space=pltpu.SMEM),
            pl.BlockSpec(memory_space=pltpu.ANY),
        ],
        out_specs=pl.BlockSpec(memory_space=pltpu.ANY),
        scratch_shapes=[
            pltpu.SemaphoreType.DMA,
            pltpu.SemaphoreType.DMA,
            pltpu.SemaphoreType.DMA,
        ],
    )(pi, x)


# baseline (device time: 390915 ns/iter reference)
import jax
import jax.numpy as jnp
from jax import lax
from jax.experimental import pallas as pl
from jax.experimental.pallas import tpu as pltpu


def kernel(x, pi):
    shard_shape = x.shape

    def body(pi_ref, x_ref, out_ref, send_sem, recv_sem, local_sem):
        my_x = lax.axis_index("x")
        my_y = lax.axis_index("y")
        dest = pi_ref[my_x]

        @pl.when(dest != my_x)
        def _swap():
            rdma = pltpu.make_async_remote_copy(
                src_ref=x_ref,
                dst_ref=out_ref,
                send_sem=send_sem,
                recv_sem=recv_sem,
                device_id=(dest, my_y),
                device_id_type=pl.DeviceIdType.MESH,
            )
            rdma.start()
            rdma.wait()

        @pl.when(dest == my_x)
        def _local():
            copy = pltpu.make_async_copy(x_ref, out_ref, local_sem)
            copy.start()
            copy.wait()

    return pl.pallas_call(
        body,
        out_shape=jax.ShapeDtypeStruct(shard_shape, jnp.float32),
        in_specs=[
            pl.BlockSpec(memory_space=pltpu.SMEM),
            pl.BlockSpec(memory_space=pl.ANY),
        ],
        out_specs=pl.BlockSpec(memory_space=pl.ANY),
        scratch_shapes=[
            pltpu.SemaphoreType.DMA,
            pltpu.SemaphoreType.DMA,
            pltpu.SemaphoreType.DMA,
        ],
    )(pi, x)


# device time: 233239 ns/iter; 1.6760x vs baseline; 1.6760x over previous
import jax
import jax.numpy as jnp
from jax import lax
from jax.experimental import pallas as pl
from jax.experimental.pallas import tpu as pltpu

N_CHUNKS = 8


def kernel(x, pi):
    shard_shape = x.shape
    half = shard_shape[1] // 2
    rows = half // N_CHUNKS

    def body(pi_ref, x_ref, out_ref,
             x_send_sems, x_recv_sems, y_send_sems, y_recv_sems, local_sem):
        my_x = lax.axis_index("x")
        my_y = lax.axis_index("y")
        dest = pi_ref[my_x]
        peer_y = 1 - my_y
        my_h0 = my_y * half
        peer_h0 = peer_y * half

        @pl.when(dest != my_x)
        def _swap():
            barrier = pltpu.get_barrier_semaphore()
            pl.semaphore_signal(barrier, inc=1, device_id=(dest, my_y),
                                device_id_type=pl.DeviceIdType.MESH)
            pl.semaphore_signal(barrier, inc=1, device_id=(my_x, peer_y),
                                device_id_type=pl.DeviceIdType.MESH)
            pl.semaphore_wait(barrier, 2)

            def x_rdma(c):
                sl = pl.ds(my_h0 + c * rows, rows)
                return pltpu.make_async_remote_copy(
                    src_ref=x_ref.at[:, sl, :],
                    dst_ref=out_ref.at[:, sl, :],
                    send_sem=x_send_sems.at[c],
                    recv_sem=x_recv_sems.at[c],
                    device_id=(dest, my_y),
                    device_id_type=pl.DeviceIdType.MESH,
                )

            def y_rdma(c):
                sl = pl.ds(my_h0 + c * rows, rows)
                return pltpu.make_async_remote_copy(
                    src_ref=out_ref.at[:, sl, :],
                    dst_ref=out_ref.at[:, sl, :],
                    send_sem=y_send_sems.at[c],
                    recv_sem=y_recv_sems.at[c],
                    device_id=(my_x, peer_y),
                    device_id_type=pl.DeviceIdType.MESH,
                )

            def y_recv(c):
                sl = pl.ds(peer_h0 + c * rows, rows)
                return pltpu.make_async_remote_copy(
                    src_ref=out_ref.at[:, sl, :],
                    dst_ref=out_ref.at[:, sl, :],
                    send_sem=y_send_sems.at[c],
                    recv_sem=y_recv_sems.at[c],
                    device_id=(my_x, peer_y),
                    device_id_type=pl.DeviceIdType.MESH,
                )

            for c in range(N_CHUNKS):
                x_rdma(c).start()

            for c in range(N_CHUNKS):
                r = x_rdma(c)
                r.wait_recv()
                y_rdma(c).start()
                r.wait_send()

            for c in range(N_CHUNKS):
                y_recv(c).wait_recv()
                y_rdma(c).wait_send()

        @pl.when(dest == my_x)
        def _local():
            copy = pltpu.make_async_copy(x_ref, out_ref, local_sem)
            copy.start()
            copy.wait()

    return pl.pallas_call(
        body,
        out_shape=jax.ShapeDtypeStruct(shard_shape, jnp.float32),
        in_specs=[
            pl.BlockSpec(memory_space=pltpu.SMEM),
            pl.BlockSpec(memory_space=pl.ANY),
        ],
        out_specs=pl.BlockSpec(memory_space=pl.ANY),
        scratch_shapes=[
            pltpu.SemaphoreType.DMA((N_CHUNKS,)),
            pltpu.SemaphoreType.DMA((N_CHUNKS,)),
            pltpu.SemaphoreType.DMA((N_CHUNKS,)),
            pltpu.SemaphoreType.DMA((N_CHUNKS,)),
            pltpu.SemaphoreType.DMA,
        ],
        compiler_params=pltpu.CompilerParams(collective_id=0),
    )(pi, x)
